# speedup vs baseline: 2.0386x; 2.0386x over previous
import functools

import jax
import jax.numpy as jnp
from jax import lax
from jax.experimental import pallas as pl
from jax.experimental.pallas import tpu as pltpu
from jax.experimental.pallas import tpu_sc as plsc

H = 1024
NW = 32
CHUNK = 64


def _sc_gather(ids_flat, word_emb, n_tokens):
    tok_per_w = n_tokens // NW
    nchunk = tok_per_w // CHUNK
    mesh = plsc.VectorSubcoreMesh(core_axis_name="c", subcore_axis_name="s")

    @functools.partial(
        pl.kernel,
        out_type=jax.ShapeDtypeStruct((n_tokens, H), jnp.float32),
        mesh=mesh,
        scratch_types=[
            pltpu.VMEM((CHUNK,), jnp.int32),
            pltpu.VMEM((CHUNK, H), jnp.float32),
            pltpu.SemaphoreType.DMA,
        ],
    )
    def gather_kernel(ids_hbm, word_hbm, out_hbm, idx_v, rows_v, sem):
        wid = lax.axis_index("s") * 2 + lax.axis_index("c")
        base = wid * tok_per_w
        for c in range(nchunk):
            off = base + c * CHUNK
            pltpu.sync_copy(ids_hbm.at[pl.ds(off, CHUNK)], idx_v)
            pltpu.async_copy(word_hbm.at[idx_v], rows_v, sem).wait()
            pltpu.sync_copy(rows_v, out_hbm.at[pl.ds(off, CHUNK)])

    return gather_kernel(ids_flat, word_emb)


def _tc_add_ln(gathered, tt_col, pos_emb, type_emb_pad, gamma2, beta2,
               n_tokens, seq_len, blk):
    grid = n_tokens // blk
    pos_blocks = seq_len // blk

    def body(g_ref, tt_ref, p_ref, te_ref, ga_ref, be_ref, o_ref):
        t = tt_ref[...]
        e0 = te_ref[0:1, :]
        e1 = te_ref[1:2, :]
        x = g_ref[...] + p_ref[...] + e0 + t * (e1 - e0)
        mean = jnp.mean(x, axis=-1, keepdims=True)
        xc = x - mean
        var = jnp.mean(xc * xc, axis=-1, keepdims=True)
        y = xc * lax.rsqrt(var + 1e-12)
        o_ref[...] = y * ga_ref[...] + be_ref[...]

    return pl.pallas_call(
        body,
        grid=(grid,),
        in_specs=[
            pl.BlockSpec((blk, H), lambda j: (j, 0)),
            pl.BlockSpec((blk, 1), lambda j: (j, 0)),
            pl.BlockSpec((blk, H), lambda j: (j % pos_blocks, 0)),
            pl.BlockSpec((8, H), lambda j: (0, 0)),
            pl.BlockSpec((1, H), lambda j: (0, 0)),
            pl.BlockSpec((1, H), lambda j: (0, 0)),
        ],
        out_specs=pl.BlockSpec((blk, H), lambda j: (j, 0)),
        out_shape=jax.ShapeDtypeStruct((n_tokens, H), jnp.float32),
    )(gathered, tt_col, pos_emb, type_emb_pad, gamma2, beta2)


def kernel(input_ids, token_type_ids, word_emb, pos_emb, type_emb,
           ln_gamma, ln_beta):
    b, s = input_ids.shape
    n_tokens = b * s
    ids_flat = input_ids.reshape(n_tokens)
    gathered = _sc_gather(ids_flat, word_emb, n_tokens)

    tt_col = token_type_ids.reshape(n_tokens, 1).astype(jnp.float32)
    type_emb_pad = jnp.concatenate(
        [type_emb, jnp.zeros((6, H), jnp.float32)], axis=0)
    gamma2 = ln_gamma.reshape(1, H)
    beta2 = ln_beta.reshape(1, H)

    out = _tc_add_ln(gathered, tt_col, pos_emb, type_emb_pad, gamma2, beta2,
                     n_tokens, s, blk=256)
    return out.reshape(b, s, H)


# speedup vs baseline: 2.0844x; 1.0224x over previous
import functools

import jax
import jax.numpy as jnp
from jax import lax
from jax.experimental import pallas as pl
from jax.experimental.pallas import tpu as pltpu
from jax.experimental.pallas import tpu_sc as plsc

H = 1024
NW = 32
CHUNK = 32


def _sc_gather(ids_flat, word_emb, n_tokens):
    tok_per_w = n_tokens // NW
    nchunk = tok_per_w // CHUNK
    mesh = plsc.VectorSubcoreMesh(core_axis_name="c", subcore_axis_name="s")

    @functools.partial(
        pl.kernel,
        out_type=jax.ShapeDtypeStruct((n_tokens, H), jnp.float32),
        mesh=mesh,
        scratch_types=[
            pltpu.VMEM((tok_per_w,), jnp.int32),
            pltpu.VMEM((CHUNK, H), jnp.float32),
            pltpu.VMEM((CHUNK, H), jnp.float32),
            pltpu.SemaphoreType.DMA,
            pltpu.SemaphoreType.DMA,
        ],
    )
    def gather_kernel(ids_hbm, word_hbm, out_hbm, idx_v, rows0, rows1,
                      gsem, osem):
        wid = lax.axis_index("s") * 2 + lax.axis_index("c")
        base = wid * tok_per_w
        pltpu.sync_copy(ids_hbm.at[pl.ds(base, tok_per_w)], idx_v)
        bufs = (rows0, rows1)

        def start_gather(c):
            return pltpu.async_copy(
                word_hbm.at[idx_v.at[pl.ds(c * CHUNK, CHUNK)]],
                bufs[c % 2], gsem)

        g = start_gather(0)
        out_prev = None
        for c in range(nchunk):
            g.wait()
            if out_prev is not None:
                out_prev.wait()
            if c + 1 < nchunk:
                g = start_gather(c + 1)
            out_prev = pltpu.async_copy(
                bufs[c % 2], out_hbm.at[pl.ds(base + c * CHUNK, CHUNK)], osem)
        out_prev.wait()

    return gather_kernel(ids_flat, word_emb)


def _tc_add_ln(gathered, tt_col, pos_emb, type_emb_pad, gamma2, beta2,
               batch, seq_len, blk):
    s_blocks = seq_len // blk
    n_tokens = batch * seq_len

    def body(g_ref, tt_ref, p_ref, te_ref, ga_ref, be_ref, o_ref):
        t = tt_ref[...]
        e0 = te_ref[0:1, :]
        e1 = te_ref[1:2, :]
        x = g_ref[...] + p_ref[...] + e0 + t * (e1 - e0)
        mean = jnp.mean(x, axis=-1, keepdims=True)
        xc = x - mean
        var = jnp.mean(xc * xc, axis=-1, keepdims=True)
        y = xc * lax.rsqrt(var + 1e-12)
        o_ref[...] = y * ga_ref[...] + be_ref[...]

    return pl.pallas_call(
        body,
        grid=(s_blocks, batch),
        in_specs=[
            pl.BlockSpec((blk, H), lambda j, b: (b * s_blocks + j, 0)),
            pl.BlockSpec((blk, 1), lambda j, b: (b * s_blocks + j, 0)),
            pl.BlockSpec((blk, H), lambda j, b: (j, 0)),
            pl.BlockSpec((8, H), lambda j, b: (0, 0)),
            pl.BlockSpec((1, H), lambda j, b: (0, 0)),
            pl.BlockSpec((1, H), lambda j, b: (0, 0)),
        ],
        out_specs=pl.BlockSpec((blk, H), lambda j, b: (b * s_blocks + j, 0)),
        out_shape=jax.ShapeDtypeStruct((n_tokens, H), jnp.float32),
    )(gathered, tt_col, pos_emb, type_emb_pad, gamma2, beta2)


def kernel(input_ids, token_type_ids, word_emb, pos_emb, type_emb,
           ln_gamma, ln_beta):
    b, s = input_ids.shape
    n_tokens = b * s
    ids_flat = input_ids.reshape(n_tokens)
    gathered = _sc_gather(ids_flat, word_emb, n_tokens)

    tt_col = token_type_ids.reshape(n_tokens, 1).astype(jnp.float32)
    type_emb_pad = jnp.concatenate(
        [type_emb, jnp.zeros((6, H), jnp.float32)], axis=0)
    gamma2 = ln_gamma.reshape(1, H)
    beta2 = ln_beta.reshape(1, H)

    out = _tc_add_ln(gathered, tt_col, pos_emb, type_emb_pad, gamma2, beta2,
                     b, s, blk=256)
    return out.reshape(b, s, H)


# speedup vs baseline: 2.3068x; 1.1067x over previous
import functools

import jax
import jax.numpy as jnp
from jax import lax
from jax.experimental import pallas as pl
from jax.experimental.pallas import tpu as pltpu
from jax.experimental.pallas import tpu_sc as plsc

H = 1024
NW = 32
CHUNK = 32


def _sc_gather(ids_flat, word_emb, n_tokens):
    tok_per_w = n_tokens // NW
    nchunk = tok_per_w // CHUNK
    mesh = plsc.VectorSubcoreMesh(core_axis_name="c", subcore_axis_name="s")

    @functools.partial(
        pl.kernel,
        out_type=jax.ShapeDtypeStruct((n_tokens, H), jnp.float32),
        mesh=mesh,
        scratch_types=[
            pltpu.VMEM((tok_per_w,), jnp.int32),
            pltpu.VMEM((CHUNK, H), jnp.float32),
            pltpu.VMEM((CHUNK, H), jnp.float32),
            pltpu.SemaphoreType.DMA,
            pltpu.SemaphoreType.DMA,
        ],
    )
    def gather_kernel(ids_hbm, word_hbm, out_hbm, idx_v, rows0, rows1,
                      gsem, osem):
        wid = lax.axis_index("s") * 2 + lax.axis_index("c")
        base = wid * tok_per_w
        pltpu.sync_copy(ids_hbm.at[pl.ds(base, tok_per_w)], idx_v)
        bufs = (rows0, rows1)

        def start_gather(c):
            return pltpu.async_copy(
                word_hbm.at[idx_v.at[pl.ds(c * CHUNK, CHUNK)]],
                bufs[c % 2], gsem)

        g = start_gather(0)
        out_prev = None
        for c in range(nchunk):
            g.wait()
            if out_prev is not None:
                out_prev.wait()
            if c + 1 < nchunk:
                g = start_gather(c + 1)
            out_prev = pltpu.async_copy(
                bufs[c % 2], out_hbm.at[pl.ds(base + c * CHUNK, CHUNK)], osem)
        out_prev.wait()

    return gather_kernel(ids_flat, word_emb)


def _tc_add_ln(gathered, tt_col, pos_emb, type_emb_pad, gamma2, beta2,
               batch, seq_len, blk):
    s_blocks = seq_len // blk
    n_tokens = batch * seq_len

    def body(g_ref, tt_ref, p_ref, te_ref, ga_ref, be_ref, o_ref):
        t = tt_ref[...]
        e0 = te_ref[0:1, :]
        e1 = te_ref[1:2, :]
        x = g_ref[...] + p_ref[...] + e0 + t * (e1 - e0)
        mean = jnp.mean(x, axis=-1, keepdims=True)
        xc = x - mean
        var = jnp.mean(xc * xc, axis=-1, keepdims=True)
        y = xc * lax.rsqrt(var + 1e-12)
        o_ref[...] = y * ga_ref[...] + be_ref[...]

    return pl.pallas_call(
        body,
        grid=(s_blocks, batch),
        in_specs=[
            pl.BlockSpec((blk, H), lambda j, b: (b * s_blocks + j, 0)),
            pl.BlockSpec((blk, 1), lambda j, b: (b * s_blocks + j, 0)),
            pl.BlockSpec((blk, H), lambda j, b: (j, 0)),
            pl.BlockSpec((8, H), lambda j, b: (0, 0)),
            pl.BlockSpec((1, H), lambda j, b: (0, 0)),
            pl.BlockSpec((1, H), lambda j, b: (0, 0)),
        ],
        out_specs=pl.BlockSpec((blk, H), lambda j, b: (b * s_blocks + j, 0)),
        out_shape=jax.ShapeDtypeStruct((n_tokens, H), jnp.float32),
    )(gathered, tt_col, pos_emb, type_emb_pad, gamma2, beta2)


def kernel(input_ids, token_type_ids, word_emb, pos_emb, type_emb,
           ln_gamma, ln_beta):
    b, s = input_ids.shape
    n_tokens = b * s
    ids_flat = input_ids.reshape(n_tokens)
    gathered = _sc_gather(ids_flat, word_emb, n_tokens)

    tt_col = token_type_ids.reshape(n_tokens, 1).astype(jnp.float32)
    type_emb_pad = jnp.concatenate(
        [type_emb, jnp.zeros((6, H), jnp.float32)], axis=0)
    gamma2 = ln_gamma.reshape(1, H)
    beta2 = ln_beta.reshape(1, H)

    out = _tc_add_ln(gathered, tt_col, pos_emb, type_emb_pad, gamma2, beta2,
                     b, s, blk=512)
    return out.reshape(b, s, H)


# speedup vs baseline: 2.4269x; 1.0521x over previous
import functools

import jax
import jax.numpy as jnp
from jax import lax
from jax.experimental import pallas as pl
from jax.experimental.pallas import tpu as pltpu
from jax.experimental.pallas import tpu_sc as plsc

H = 1024
NW = 32
CHUNK = 32


def _sc_gather(ids_flat, word_emb, n_tokens):
    tok_per_w = n_tokens // NW
    nchunk = tok_per_w // CHUNK
    mesh = plsc.VectorSubcoreMesh(core_axis_name="c", subcore_axis_name="s")

    @functools.partial(
        pl.kernel,
        out_type=jax.ShapeDtypeStruct((n_tokens, H), jnp.float32),
        mesh=mesh,
        scratch_types=[
            pltpu.VMEM((tok_per_w,), jnp.int32),
            pltpu.VMEM((CHUNK, H), jnp.float32),
            pltpu.VMEM((CHUNK, H), jnp.float32),
            pltpu.SemaphoreType.DMA,
            pltpu.SemaphoreType.DMA,
        ],
    )
    def gather_kernel(ids_hbm, word_hbm, out_hbm, idx_v, rows0, rows1,
                      gsem, osem):
        wid = lax.axis_index("s") * 2 + lax.axis_index("c")
        base = wid * tok_per_w
        pltpu.sync_copy(ids_hbm.at[pl.ds(base, tok_per_w)], idx_v)
        bufs = (rows0, rows1)

        def start_gather(c):
            return pltpu.async_copy(
                word_hbm.at[idx_v.at[pl.ds(c * CHUNK, CHUNK)]],
                bufs[c % 2], gsem)

        g = start_gather(0)
        out_prev = None
        for c in range(nchunk):
            g.wait()
            if out_prev is not None:
                out_prev.wait()
            if c + 1 < nchunk:
                g = start_gather(c + 1)
            out_prev = pltpu.async_copy(
                bufs[c % 2], out_hbm.at[pl.ds(base + c * CHUNK, CHUNK)], osem)
        out_prev.wait()

    return gather_kernel(ids_flat, word_emb)


def _tc_add_ln(gathered, tt_col, pos_emb, type_emb_pad, gamma2, beta2,
               batch, seq_len, blk):
    s_blocks = seq_len // blk
    n_tokens = batch * seq_len

    def body(g_ref, tt_ref, p_ref, te_ref, ga_ref, be_ref, o_ref):
        t = tt_ref[...]
        e0 = te_ref[0:1, :]
        e1 = te_ref[1:2, :]
        x = g_ref[...] + p_ref[...] + e0 + t * (e1 - e0)
        mean = jnp.mean(x, axis=-1, keepdims=True)
        xc = x - mean
        var = jnp.mean(xc * xc, axis=-1, keepdims=True)
        y = xc * lax.rsqrt(var + 1e-12)
        o_ref[...] = y * ga_ref[...] + be_ref[...]

    return pl.pallas_call(
        body,
        grid=(s_blocks, batch),
        in_specs=[
            pl.BlockSpec((blk, H), lambda j, b: (b * s_blocks + j, 0)),
            pl.BlockSpec((blk, 1), lambda j, b: (b * s_blocks + j, 0)),
            pl.BlockSpec((blk, H), lambda j, b: (j, 0)),
            pl.BlockSpec((8, H), lambda j, b: (0, 0)),
            pl.BlockSpec((1, H), lambda j, b: (0, 0)),
            pl.BlockSpec((1, H), lambda j, b: (0, 0)),
        ],
        out_specs=pl.BlockSpec((blk, H), lambda j, b: (b * s_blocks + j, 0)),
        out_shape=jax.ShapeDtypeStruct((n_tokens, H), jnp.float32),
    )(gathered, tt_col, pos_emb, type_emb_pad, gamma2, beta2)


def kernel(input_ids, token_type_ids, word_emb, pos_emb, type_emb,
           ln_gamma, ln_beta):
    b, s = input_ids.shape
    n_tokens = b * s
    ids_flat = input_ids.reshape(n_tokens)
    gathered = _sc_gather(ids_flat, word_emb, n_tokens)

    tt_col = token_type_ids.reshape(n_tokens, 1).astype(jnp.float32)
    type_emb_pad = jnp.concatenate(
        [type_emb, jnp.zeros((6, H), jnp.float32)], axis=0)
    gamma2 = ln_gamma.reshape(1, H)
    beta2 = ln_beta.reshape(1, H)

    out = _tc_add_ln(gathered, tt_col, pos_emb, type_emb_pad, gamma2, beta2,
                     b, s, blk=1024)
    return out.reshape(b, s, H)


# speedup vs baseline: 2.4919x; 1.0268x over previous
import functools

import jax
import jax.numpy as jnp
from jax import lax
from jax.experimental import pallas as pl
from jax.experimental.pallas import tpu as pltpu
from jax.experimental.pallas import tpu_sc as plsc

H = 1024
NW = 32
CHUNK = 32


def _sc_gather(ids_flat, word_emb, n_tokens):
    tok_per_w = n_tokens // NW
    nchunk = tok_per_w // CHUNK
    mesh = plsc.VectorSubcoreMesh(core_axis_name="c", subcore_axis_name="s")

    @functools.partial(
        pl.kernel,
        out_type=jax.ShapeDtypeStruct((n_tokens, H), jnp.float32),
        mesh=mesh,
        scratch_types=[
            pltpu.VMEM((tok_per_w,), jnp.int32),
            pltpu.VMEM((CHUNK, H), jnp.float32),
            pltpu.VMEM((CHUNK, H), jnp.float32),
            pltpu.VMEM((CHUNK, H), jnp.float32),
            pltpu.SemaphoreType.DMA,
            pltpu.SemaphoreType.DMA,
            pltpu.SemaphoreType.DMA,
            pltpu.SemaphoreType.DMA,
        ],
    )
    def gather_kernel(ids_hbm, word_hbm, out_hbm, idx_v, rows0, rows1, rows2,
                      gsem0, gsem1, osem0, osem1):
        wid = lax.axis_index("s") * 2 + lax.axis_index("c")
        base = wid * tok_per_w
        pltpu.sync_copy(ids_hbm.at[pl.ds(base, tok_per_w)], idx_v)
        bufs = (rows0, rows1, rows2)
        gsems = (gsem0, gsem1)
        osems = (osem0, osem1)

        def start_gather(c):
            return pltpu.async_copy(
                word_hbm.at[idx_v.at[pl.ds(c * CHUNK, CHUNK)]],
                bufs[c % 3], gsems[c % 2])

        def start_out(c):
            return pltpu.async_copy(
                bufs[c % 3], out_hbm.at[pl.ds(base + c * CHUNK, CHUNK)],
                osems[c % 2])

        gathers = [None] * nchunk
        outs = [None] * nchunk
        gathers[0] = start_gather(0)
        if nchunk > 1:
            gathers[1] = start_gather(1)
        for c in range(nchunk):
            gathers[c].wait()
            if c >= 1:
                outs[c - 1].wait()
            if c + 2 < nchunk:
                gathers[c + 2] = start_gather(c + 2)
            outs[c] = start_out(c)
        outs[nchunk - 1].wait()

    return gather_kernel(ids_flat, word_emb)


def _tc_add_ln(gathered, tt_col, pos_emb, type_emb_pad, gamma2, beta2,
               batch, seq_len, blk):
    s_blocks = seq_len // blk
    n_tokens = batch * seq_len

    def body(g_ref, tt_ref, p_ref, te_ref, ga_ref, be_ref, o_ref):
        t = tt_ref[...]
        e0 = te_ref[0:1, :]
        e1 = te_ref[1:2, :]
        x = g_ref[...] + p_ref[...] + e0 + t * (e1 - e0)
        mean = jnp.mean(x, axis=-1, keepdims=True)
        xc = x - mean
        var = jnp.mean(xc * xc, axis=-1, keepdims=True)
        y = xc * lax.rsqrt(var + 1e-12)
        o_ref[...] = y * ga_ref[...] + be_ref[...]

    return pl.pallas_call(
        body,
        grid=(s_blocks, batch),
        in_specs=[
            pl.BlockSpec((blk, H), lambda j, b: (b * s_blocks + j, 0)),
            pl.BlockSpec((blk, 1), lambda j, b: (b * s_blocks + j, 0)),
            pl.BlockSpec((blk, H), lambda j, b: (j, 0)),
            pl.BlockSpec((8, H), lambda j, b: (0, 0)),
            pl.BlockSpec((1, H), lambda j, b: (0, 0)),
            pl.BlockSpec((1, H), lambda j, b: (0, 0)),
        ],
        out_specs=pl.BlockSpec((blk, H), lambda j, b: (b * s_blocks + j, 0)),
        out_shape=jax.ShapeDtypeStruct((n_tokens, H), jnp.float32),
    )(gathered, tt_col, pos_emb, type_emb_pad, gamma2, beta2)


def kernel(input_ids, token_type_ids, word_emb, pos_emb, type_emb,
           ln_gamma, ln_beta):
    b, s = input_ids.shape
    n_tokens = b * s
    ids_flat = input_ids.reshape(n_tokens)
    gathered = _sc_gather(ids_flat, word_emb, n_tokens)

    tt_col = token_type_ids.reshape(n_tokens, 1).astype(jnp.float32)
    type_emb_pad = jnp.concatenate(
        [type_emb, jnp.zeros((6, H), jnp.float32)], axis=0)
    gamma2 = ln_gamma.reshape(1, H)
    beta2 = ln_beta.reshape(1, H)

    out = _tc_add_ln(gathered, tt_col, pos_emb, type_emb_pad, gamma2, beta2,
                     b, s, blk=1024)
    return out.reshape(b, s, H)


# speedup vs baseline: 2.5312x; 1.0158x over previous
import functools

import jax
import jax.numpy as jnp
from jax import lax
from jax.experimental import pallas as pl
from jax.experimental.pallas import tpu as pltpu
from jax.experimental.pallas import tpu_sc as plsc

H = 1024
NW = 32
CHUNK = 32


def _sc_gather(ids_flat, word_emb, n_tokens):
    tok_per_w = n_tokens // NW
    nchunk = tok_per_w // CHUNK
    mesh = plsc.VectorSubcoreMesh(core_axis_name="c", subcore_axis_name="s")

    @functools.partial(
        pl.kernel,
        out_type=jax.ShapeDtypeStruct((n_tokens, H), jnp.float32),
        mesh=mesh,
        scratch_types=[
            pltpu.VMEM((tok_per_w,), jnp.int32),
            pltpu.VMEM((CHUNK, H), jnp.float32),
            pltpu.VMEM((CHUNK, H), jnp.float32),
            pltpu.VMEM((CHUNK, H), jnp.float32),
            pltpu.SemaphoreType.DMA,
            pltpu.SemaphoreType.DMA,
            pltpu.SemaphoreType.DMA,
            pltpu.SemaphoreType.DMA,
        ],
    )
    def gather_kernel(ids_hbm, word_hbm, out_hbm, idx_v, rows0, rows1, rows2,
                      gsem0, gsem1, osem0, osem1):
        wid = lax.axis_index("s") * 2 + lax.axis_index("c")
        base = wid * tok_per_w
        pltpu.sync_copy(ids_hbm.at[pl.ds(base, tok_per_w)], idx_v)
        bufs = (rows0, rows1, rows2)
        gsems = (gsem0, gsem1)
        osems = (osem0, osem1)

        def start_gather(c):
            return pltpu.async_copy(
                word_hbm.at[idx_v.at[pl.ds(c * CHUNK, CHUNK)]],
                bufs[c % 3], gsems[c % 2])

        def start_out(c):
            return pltpu.async_copy(
                bufs[c % 3], out_hbm.at[pl.ds(base + c * CHUNK, CHUNK)],
                osems[c % 2])

        gathers = [None] * nchunk
        outs = [None] * nchunk
        gathers[0] = start_gather(0)
        if nchunk > 1:
            gathers[1] = start_gather(1)
        for c in range(nchunk):
            gathers[c].wait()
            if c >= 1:
                outs[c - 1].wait()
            if c + 2 < nchunk:
                gathers[c + 2] = start_gather(c + 2)
            outs[c] = start_out(c)
        outs[nchunk - 1].wait()

    return gather_kernel(ids_flat, word_emb)


def _tc_add_ln(gathered, tt_col, pos_emb, type_emb_pad, gamma2, beta2,
               batch, seq_len, blk):
    s_blocks = seq_len // blk
    n_tokens = batch * seq_len

    def body(g_ref, tt_ref, p_ref, te_ref, ga_ref, be_ref, o_ref):
        t = tt_ref[...]
        e0 = te_ref[0:1, :]
        e1 = te_ref[1:2, :]
        x = g_ref[...] + p_ref[...] + e0 + t * (e1 - e0)
        mean = jnp.mean(x, axis=-1, keepdims=True)
        xc = x - mean
        var = jnp.mean(xc * xc, axis=-1, keepdims=True)
        y = xc * lax.rsqrt(var + 1e-12)
        o_ref[...] = y * ga_ref[...] + be_ref[...]

    return pl.pallas_call(
        body,
        grid=(s_blocks, batch),
        in_specs=[
            pl.BlockSpec((blk, H), lambda j, b: (b * s_blocks + j, 0)),
            pl.BlockSpec((blk, 1), lambda j, b: (b * s_blocks + j, 0)),
            pl.BlockSpec((blk, H), lambda j, b: (j, 0)),
            pl.BlockSpec((8, H), lambda j, b: (0, 0)),
            pl.BlockSpec((1, H), lambda j, b: (0, 0)),
            pl.BlockSpec((1, H), lambda j, b: (0, 0)),
        ],
        out_specs=pl.BlockSpec((blk, H), lambda j, b: (b * s_blocks + j, 0)),
        out_shape=jax.ShapeDtypeStruct((n_tokens, H), jnp.float32),
    )(gathered, tt_col, pos_emb, type_emb_pad, gamma2, beta2)


def kernel(input_ids, token_type_ids, word_emb, pos_emb, type_emb,
           ln_gamma, ln_beta):
    b, s = input_ids.shape
    n_tokens = b * s
    ids_flat = input_ids.reshape(n_tokens)
    gathered = _sc_gather(ids_flat, word_emb, n_tokens)

    tt_col = token_type_ids.reshape(n_tokens, 1).astype(jnp.float32)
    type_emb_pad = jnp.concatenate(
        [type_emb, jnp.zeros((6, H), jnp.float32)], axis=0)
    gamma2 = ln_gamma.reshape(1, H)
    beta2 = ln_beta.reshape(1, H)

    out = _tc_add_ln(gathered, tt_col, pos_emb, type_emb_pad, gamma2, beta2,
                     b, s, blk=2048)
    return out.reshape(b, s, H)


# speedup vs baseline: 2.5461x; 1.0059x over previous
import functools

import jax
import jax.numpy as jnp
from jax import lax
from jax.experimental import pallas as pl
from jax.experimental.pallas import tpu as pltpu
from jax.experimental.pallas import tpu_sc as plsc

H = 1024
NW = 32
CHUNK = 32


def _sc_gather(ids_flat, word_emb, n_tokens):
    tok_per_w = n_tokens // NW
    nchunk = tok_per_w // CHUNK
    mesh = plsc.VectorSubcoreMesh(core_axis_name="c", subcore_axis_name="s")

    @functools.partial(
        pl.kernel,
        out_type=jax.ShapeDtypeStruct((n_tokens, H), jnp.float32),
        mesh=mesh,
        scratch_types=[
            pltpu.VMEM((tok_per_w,), jnp.int32),
            pltpu.VMEM((CHUNK, H), jnp.float32),
            pltpu.VMEM((CHUNK, H), jnp.float32),
            pltpu.VMEM((CHUNK, H), jnp.float32),
            pltpu.SemaphoreType.DMA,
            pltpu.SemaphoreType.DMA,
            pltpu.SemaphoreType.DMA,
            pltpu.SemaphoreType.DMA,
        ],
    )
    def gather_kernel(ids_hbm, word_hbm, out_hbm, idx_v, rows0, rows1, rows2,
                      gsem0, gsem1, osem0, osem1):
        wid = lax.axis_index("s") * 2 + lax.axis_index("c")
        base = wid * tok_per_w
        pltpu.sync_copy(ids_hbm.at[pl.ds(base, tok_per_w)], idx_v)
        bufs = (rows0, rows1, rows2)
        gsems = (gsem0, gsem1)
        osems = (osem0, osem1)

        def start_gather(c):
            return pltpu.async_copy(
                word_hbm.at[idx_v.at[pl.ds(c * CHUNK, CHUNK)]],
                bufs[c % 3], gsems[c % 2])

        def start_out(c):
            return pltpu.async_copy(
                bufs[c % 3], out_hbm.at[pl.ds(base + c * CHUNK, CHUNK)],
                osems[c % 2])

        gathers = [None] * nchunk
        outs = [None] * nchunk
        gathers[0] = start_gather(0)
        if nchunk > 1:
            gathers[1] = start_gather(1)
        for c in range(nchunk):
            gathers[c].wait()
            if c >= 1:
                outs[c - 1].wait()
            if c + 2 < nchunk:
                gathers[c + 2] = start_gather(c + 2)
            outs[c] = start_out(c)
        outs[nchunk - 1].wait()

    return gather_kernel(ids_flat, word_emb)


def _tc_add_ln_chunk(gathered_k, tt_col_k, pos_emb, type_emb_pad, gamma2,
                     beta2, out_prev, batch, seq_len, k, num_chunks):
    blk = seq_len // num_chunks
    n_tokens = batch * seq_len
    sb = seq_len // blk

    def body(*refs):
        if k == 0:
            g_ref, tt_ref, p_ref, te_ref, ga_ref, be_ref, o_ref = refs
        else:
            _, g_ref, tt_ref, p_ref, te_ref, ga_ref, be_ref, o_ref = refs
        t = tt_ref[...]
        e0 = te_ref[0:1, :]
        e1 = te_ref[1:2, :]
        x = g_ref[...] + p_ref[...] + e0 + t * (e1 - e0)
        mean = jnp.mean(x, axis=-1, keepdims=True)
        xc = x - mean
        var = jnp.mean(xc * xc, axis=-1, keepdims=True)
        y = xc * lax.rsqrt(var + 1e-12)
        o_ref[...] = y * ga_ref[...] + be_ref[...]

    chunk_specs = [
        pl.BlockSpec((blk, H), lambda b: (b, 0)),
        pl.BlockSpec((blk, 1), lambda b: (b, 0)),
        pl.BlockSpec((blk, H), lambda b: (k, 0)),
        pl.BlockSpec((8, H), lambda b: (0, 0)),
        pl.BlockSpec((1, H), lambda b: (0, 0)),
        pl.BlockSpec((1, H), lambda b: (0, 0)),
    ]
    out_spec = pl.BlockSpec((blk, H), lambda b: (b * sb + k, 0))
    if k == 0:
        return pl.pallas_call(
            body,
            grid=(batch,),
            in_specs=chunk_specs,
            out_specs=out_spec,
            out_shape=jax.ShapeDtypeStruct((n_tokens, H), jnp.float32),
        )(gathered_k, tt_col_k, pos_emb, type_emb_pad, gamma2, beta2)
    return pl.pallas_call(
        body,
        grid=(batch,),
        in_specs=[pl.BlockSpec(memory_space=pl.ANY)] + chunk_specs,
        out_specs=out_spec,
        out_shape=jax.ShapeDtypeStruct((n_tokens, H), jnp.float32),
        input_output_aliases={0: 0},
    )(out_prev, gathered_k, tt_col_k, pos_emb, type_emb_pad, gamma2, beta2)


def kernel(input_ids, token_type_ids, word_emb, pos_emb, type_emb,
           ln_gamma, ln_beta):
    b, s = input_ids.shape
    num_chunks = 2
    sk = s // num_chunks

    type_emb_pad = jnp.concatenate(
        [type_emb, jnp.zeros((6, H), jnp.float32)], axis=0)
    gamma2 = ln_gamma.reshape(1, H)
    beta2 = ln_beta.reshape(1, H)

    gathered = []
    tt_cols = []
    for k in range(num_chunks):
        ids_k = lax.slice(input_ids, (0, k * sk), (b, (k + 1) * sk))
        gathered.append(_sc_gather(ids_k.reshape(b * sk), word_emb, b * sk))
        tt_k = lax.slice(token_type_ids, (0, k * sk), (b, (k + 1) * sk))
        tt_cols.append(tt_k.reshape(b * sk, 1).astype(jnp.float32))

    out = None
    for k in range(num_chunks):
        out = _tc_add_ln_chunk(gathered[k], tt_cols[k], pos_emb,
                               type_emb_pad, gamma2, beta2, out,
                               b, s, k, num_chunks)
    return out.reshape(b, s, H)
